# baseline (speedup 1.0000x reference)
import jax
import jax.numpy as jnp
from jax.experimental import pallas as pl
from jax.experimental.pallas import tpu as pltpu

_ANCHORS = 3
_HEIGHT = 256
_WIDTH = 256
_OUT_P = 1000
_IOU_THR = 0.5
_RATIO = 0.3
_SHAPES = [(128, 128), (64, 64), (32, 32)]
_B = 4
_C = 64
_TOTAL = sum(h * w for h, w in _SHAPES) * _ANCHORS
_TOPK = int(_TOTAL * _RATIO)
_ROWS = 152
_PAD = _ROWS * 128
_BIG = 2**30
_BOUND1 = _SHAPES[0][0] * _SHAPES[0][1] * _ANCHORS
_BOUND2 = _BOUND1 + _SHAPES[1][0] * _SHAPES[1][1] * _ANCHORS


def _conv3x3(x, w, b):
    out = jax.lax.conv_general_dilated(x, w, (1, 1), 'SAME',
                                       dimension_numbers=('NHWC', 'HWIO', 'NHWC'))
    return out + b


def _conv1x1(x, w, b):
    return jnp.einsum('bhwc,co->bhwo', x, w[0, 0]) + b


def _sig_kernel(x_ref, out_ref):
    out_ref[...] = jax.nn.sigmoid(x_ref[...])


def _sig(x):
    return pl.pallas_call(
        _sig_kernel,
        out_shape=jax.ShapeDtypeStruct(x.shape, jnp.float32),
    )(x)


def _nms_kernel(t_ref, g_ref, fi_ref, planes_ref, valid_ref, kept_ref):
    g = g_ref[0]
    is1 = g >= _BOUND1
    is2 = g >= _BOUND2
    base = jnp.where(is2, _BOUND2, jnp.where(is1, _BOUND1, 0))
    p = (g - base).astype(jnp.float32)
    wf = jnp.where(is2, 32.0, jnp.where(is1, 64.0, 128.0)).astype(jnp.float32)
    stride = jnp.where(is2, 8.0, jnp.where(is1, 4.0, 2.0)).astype(jnp.float32)
    cell = jnp.floor(p / 3.0)
    row = jnp.floor(cell / wf)
    col = cell - row * wf
    cx = col * stride
    cy = row * stride
    t0 = t_ref[0, 0]
    t1 = t_ref[0, 1]
    t2 = t_ref[0, 2]
    t3 = t_ref[0, 3]
    img = jnp.float32(_HEIGHT)
    xy0 = t0 * img + cx
    xy1 = t1 * img + cy
    hw0 = (jnp.exp(t2) * img + stride) / 2.0
    hw1 = (jnp.exp(t3) * img + stride) / 2.0
    c0 = jnp.minimum(jnp.maximum(xy0 - hw0, 0.0), img)
    c1 = jnp.minimum(jnp.maximum(xy1 - hw1, 0.0), img)
    c2 = jnp.minimum(jnp.maximum(xy0 + hw0, 0.0), img)
    c3 = jnp.minimum(jnp.maximum(xy1 + hw1, 0.0), img)
    planes_ref[0, 0] = c0
    planes_ref[0, 1] = c1
    planes_ref[0, 2] = c2
    planes_ref[0, 3] = c3

    area = jnp.maximum(c2 - c0, 0.0) * jnp.maximum(c3 - c1, 0.0)
    ridx = jax.lax.broadcasted_iota(jnp.int32, (_ROWS, 128), 0)
    cidx = jax.lax.broadcasted_iota(jnp.int32, (_ROWS, 128), 1)
    flat = ridx * 128 + cidx
    valid_ref[...] = (flat < _TOPK).astype(jnp.float32)
    kept_ref[...] = jnp.zeros((8, 128), jnp.int32)
    oj = jax.lax.broadcasted_iota(jnp.int32, (8, 128), 0) * 128 + \
        jax.lax.broadcasted_iota(jnp.int32, (8, 128), 1)

    def cond(carry):
        fv, n_kept, last = carry
        return (fv < _BIG) & (n_kept < _OUT_P)

    def body(carry):
        fv, n_kept, last = carry
        onehot = flat == fv
        neg = jnp.float32(-1e30)
        b0 = jnp.max(jnp.where(onehot, c0, neg))
        b1 = jnp.max(jnp.where(onehot, c1, neg))
        b2 = jnp.max(jnp.where(onehot, c2, neg))
        b3 = jnp.max(jnp.where(onehot, c3, neg))
        area_b = jnp.maximum(b2 - b0, 0.0) * jnp.maximum(b3 - b1, 0.0)
        yy1 = jnp.maximum(b0, c0)
        xx1 = jnp.maximum(b1, c1)
        yy2 = jnp.minimum(b2, c2)
        xx2 = jnp.minimum(b3, c3)
        inter = jnp.maximum(yy2 - yy1, 0.0) * jnp.maximum(xx2 - xx1, 0.0)
        union = area_b + area - inter
        iou = jnp.where(union > 0.0, inter / union, 0.0)
        v = valid_ref[...] > 0.5
        v = v & (iou <= _IOU_THR) & jnp.logical_not(onehot)
        valid_ref[...] = v.astype(jnp.float32)
        kept_ref[...] = kept_ref[...] + jnp.where(oj == n_kept, fv, 0)
        fv_new = jnp.min(jnp.where(v, flat, _BIG))
        return fv_new, n_kept + 1, fv

    _, n_kept, last = jax.lax.while_loop(cond, body, (jnp.int32(0), jnp.int32(0), jnp.int32(0)))

    additional = _OUT_P - n_kept
    starting = jnp.minimum(_TOPK - additional, last + 1)
    fi = jnp.where(oj < n_kept, kept_ref[...], starting + (oj - n_kept))
    fi_ref[0] = jnp.clip(fi, 0, _TOPK - 1)


def _nms(t_planes, gidx):
    B = t_planes.shape[0]
    fi, planes = pl.pallas_call(
        _nms_kernel,
        grid=(B,),
        in_specs=[pl.BlockSpec((1, 4, _ROWS, 128), lambda b: (b, 0, 0, 0)),
                  pl.BlockSpec((1, _ROWS, 128), lambda b: (b, 0, 0))],
        out_specs=[pl.BlockSpec((1, 8, 128), lambda b: (b, 0, 0)),
                   pl.BlockSpec((1, 4, _ROWS, 128), lambda b: (b, 0, 0, 0))],
        out_shape=[jax.ShapeDtypeStruct((B, 8, 128), jnp.int32),
                   jax.ShapeDtypeStruct((B, 4, _ROWS, 128), jnp.float32)],
        scratch_shapes=[pltpu.VMEM((_ROWS, 128), jnp.float32),
                        pltpu.VMEM((8, 128), jnp.int32)],
    )(t_planes, gidx)
    return fi.reshape(B, 1024), planes


def kernel(feat0, feat1, feat2, W_in0, b_in0, W_bb0, b_bb0, W_cf0, b_cf0,
           W_in1, b_in1, W_bb1, b_bb1, W_cf1, b_cf1,
           W_in2, b_in2, W_bb2, b_bb2, W_cf2, b_cf2):
    feats = [feat0, feat1, feat2]
    Wi = [W_in0, W_in1, W_in2]
    bi = [b_in0, b_in1, b_in2]
    Wb = [W_bb0, W_bb1, W_bb2]
    bb = [b_bb0, b_bb1, b_bb2]
    Wc = [W_cf0, W_cf1, W_cf2]
    bc = [b_cf0, b_cf1, b_cf2]
    confs = []
    tplanes = [[] for _ in range(4)]
    for s, (H, W) in enumerate(_SHAPES):
        f = jax.nn.relu(_conv3x3(feats[s], Wi[s], bi[s]))
        c = _sig(_conv1x1(f, Wc[s], bc[s]).reshape(_B, H * W * _ANCHORS))
        confs.append(c)
        for cc in range(4):
            wz = Wb[s][:, :, :, cc::4]
            bz = bb[s][cc::4]
            t = _conv1x1(f, wz, bz).reshape(_B, H * W * _ANCHORS)
            tplanes[cc].append(t)
    conf = jnp.concatenate(confs, axis=-1)
    tps = [jnp.concatenate(tp, axis=-1) for tp in tplanes]

    vals, idx = jax.lax.top_k(conf, _TOPK)
    tg = [jnp.take_along_axis(tp, idx, axis=1) for tp in tps]
    npad = _PAD - _TOPK
    tg = [jnp.pad(t, ((0, 0), (0, npad))).reshape(_B, 1, _ROWS, 128) for t in tg]
    t_planes = jnp.concatenate(tg, axis=1)
    gidx = jnp.pad(idx, ((0, 0), (0, npad))).reshape(_B, _ROWS, 128)

    fi, planes = _nms(t_planes, gidx)
    fi = fi[:, :_OUT_P]
    conf_out = jnp.take_along_axis(vals, fi, axis=1)
    pf = planes.reshape(_B, 4, _PAD)
    box_out = jnp.stack([jnp.take_along_axis(pf[:, cc], fi, axis=1)
                         for cc in range(4)], axis=-1)
    return conf_out, box_out


# speedup vs baseline: 3.5580x; 3.5580x over previous
import jax
import jax.numpy as jnp
from jax.experimental import pallas as pl
from jax.experimental.pallas import tpu as pltpu

_ANCHORS = 3
_HEIGHT = 256
_WIDTH = 256
_OUT_P = 1000
_IOU_THR = 0.5
_RATIO = 0.3
_SHAPES = [(128, 128), (64, 64), (32, 32)]
_B = 4
_C = 64
_TOTAL = sum(h * w for h, w in _SHAPES) * _ANCHORS
_TOPK = int(_TOTAL * _RATIO)
_ROWS = 152
_PAD = _ROWS * 128
_BIG = 2**30
_BOUND1 = _SHAPES[0][0] * _SHAPES[0][1] * _ANCHORS
_BOUND2 = _BOUND1 + _SHAPES[1][0] * _SHAPES[1][1] * _ANCHORS


def _conv3x3(x, w, b):
    out = jax.lax.conv_general_dilated(x, w, (1, 1), 'SAME',
                                       dimension_numbers=('NHWC', 'HWIO', 'NHWC'))
    return out + b


def _conv1x1(x, w, b):
    return jnp.einsum('bhwc,co->bhwo', x, w[0, 0]) + b


def _sig_kernel(x_ref, out_ref):
    out_ref[...] = jax.nn.sigmoid(x_ref[...])


def _sig(x):
    return pl.pallas_call(
        _sig_kernel,
        out_shape=jax.ShapeDtypeStruct(x.shape, jnp.float32),
    )(x)


def _nms_kernel(t_ref, g_ref, fi_ref, planes_ref, valid_ref, kept_ref):
    g = g_ref[0]
    is1 = g >= _BOUND1
    is2 = g >= _BOUND2
    base = jnp.where(is2, _BOUND2, jnp.where(is1, _BOUND1, 0))
    p = (g - base).astype(jnp.float32)
    wf = jnp.where(is2, 32.0, jnp.where(is1, 64.0, 128.0)).astype(jnp.float32)
    stride = jnp.where(is2, 8.0, jnp.where(is1, 4.0, 2.0)).astype(jnp.float32)
    cell = jnp.floor(p / 3.0)
    row = jnp.floor(cell / wf)
    col = cell - row * wf
    cx = col * stride
    cy = row * stride
    t0 = t_ref[0, 0]
    t1 = t_ref[0, 1]
    t2 = t_ref[0, 2]
    t3 = t_ref[0, 3]
    img = jnp.float32(_HEIGHT)
    xy0 = t0 * img + cx
    xy1 = t1 * img + cy
    hw0 = (jnp.exp(t2) * img + stride) / 2.0
    hw1 = (jnp.exp(t3) * img + stride) / 2.0
    c0 = jnp.minimum(jnp.maximum(xy0 - hw0, 0.0), img)
    c1 = jnp.minimum(jnp.maximum(xy1 - hw1, 0.0), img)
    c2 = jnp.minimum(jnp.maximum(xy0 + hw0, 0.0), img)
    c3 = jnp.minimum(jnp.maximum(xy1 + hw1, 0.0), img)
    planes_ref[0, 0] = c0
    planes_ref[0, 1] = c1
    planes_ref[0, 2] = c2
    planes_ref[0, 3] = c3

    area = jnp.maximum(c2 - c0, 0.0) * jnp.maximum(c3 - c1, 0.0)
    ridx = jax.lax.broadcasted_iota(jnp.int32, (_ROWS, 128), 0)
    cidx = jax.lax.broadcasted_iota(jnp.int32, (_ROWS, 128), 1)
    flat = ridx * 128 + cidx
    valid_ref[...] = (flat < _TOPK).astype(jnp.float32)
    kept_ref[...] = jnp.zeros((8, 128), jnp.int32)
    oj = jax.lax.broadcasted_iota(jnp.int32, (8, 128), 0) * 128 + \
        jax.lax.broadcasted_iota(jnp.int32, (8, 128), 1)

    def cond(carry):
        fv, n_kept, last = carry
        return (fv < _BIG) & (n_kept < _OUT_P)

    def body(carry):
        fv, n_kept, last = carry
        onehot = flat == fv
        neg = jnp.float32(-1e30)
        b0 = jnp.max(jnp.where(onehot, c0, neg))
        b1 = jnp.max(jnp.where(onehot, c1, neg))
        b2 = jnp.max(jnp.where(onehot, c2, neg))
        b3 = jnp.max(jnp.where(onehot, c3, neg))
        area_b = jnp.maximum(b2 - b0, 0.0) * jnp.maximum(b3 - b1, 0.0)
        yy1 = jnp.maximum(b0, c0)
        xx1 = jnp.maximum(b1, c1)
        yy2 = jnp.minimum(b2, c2)
        xx2 = jnp.minimum(b3, c3)
        inter = jnp.maximum(yy2 - yy1, 0.0) * jnp.maximum(xx2 - xx1, 0.0)
        union = area_b + area - inter
        iou = jnp.where(union > 0.0, inter / union, 0.0)
        v = valid_ref[...] > 0.5
        v = v & (iou <= _IOU_THR) & jnp.logical_not(onehot)
        valid_ref[...] = v.astype(jnp.float32)
        kept_ref[...] = kept_ref[...] + jnp.where(oj == n_kept, fv, 0)
        fv_new = jnp.min(jnp.where(v, flat, _BIG))
        return fv_new, n_kept + 1, fv

    _, n_kept, last = jax.lax.while_loop(cond, body, (jnp.int32(0), jnp.int32(0), jnp.int32(0)))

    additional = _OUT_P - n_kept
    starting = jnp.minimum(_TOPK - additional, last + 1)
    fi = jnp.where(oj < n_kept, kept_ref[...], starting + (oj - n_kept))
    fi_ref[0] = jnp.clip(fi, 0, _TOPK - 1)


def _nms(t_planes, gidx):
    B = t_planes.shape[0]
    fi, planes = pl.pallas_call(
        _nms_kernel,
        grid=(B,),
        in_specs=[pl.BlockSpec((1, 4, _ROWS, 128), lambda b: (b, 0, 0, 0)),
                  pl.BlockSpec((1, _ROWS, 128), lambda b: (b, 0, 0))],
        out_specs=[pl.BlockSpec((1, 8, 128), lambda b: (b, 0, 0)),
                   pl.BlockSpec((1, 4, _ROWS, 128), lambda b: (b, 0, 0, 0))],
        out_shape=[jax.ShapeDtypeStruct((B, 8, 128), jnp.int32),
                   jax.ShapeDtypeStruct((B, 4, _ROWS, 128), jnp.float32)],
        scratch_shapes=[pltpu.VMEM((_ROWS, 128), jnp.float32),
                        pltpu.VMEM((8, 128), jnp.int32)],
    )(t_planes, gidx)
    return fi.reshape(B, 1024), planes


def kernel(feat0, feat1, feat2, W_in0, b_in0, W_bb0, b_bb0, W_cf0, b_cf0,
           W_in1, b_in1, W_bb1, b_bb1, W_cf1, b_cf1,
           W_in2, b_in2, W_bb2, b_bb2, W_cf2, b_cf2):
    feats = [feat0, feat1, feat2]
    Wi = [W_in0, W_in1, W_in2]
    bi = [b_in0, b_in1, b_in2]
    Wb = [W_bb0, W_bb1, W_bb2]
    bb = [b_bb0, b_bb1, b_bb2]
    Wc = [W_cf0, W_cf1, W_cf2]
    bc = [b_cf0, b_cf1, b_cf2]
    confs = []
    tplanes = [[] for _ in range(4)]
    for s, (H, W) in enumerate(_SHAPES):
        f = jax.nn.relu(_conv3x3(feats[s], Wi[s], bi[s]))
        c = _sig(_conv1x1(f, Wc[s], bc[s]).reshape(_B, H * W * _ANCHORS))
        confs.append(c)
        for cc in range(4):
            wz = Wb[s][:, :, :, cc::4]
            bz = bb[s][cc::4]
            t = _conv1x1(f, wz, bz).reshape(_B, H * W * _ANCHORS)
            tplanes[cc].append(t)
    conf = jnp.concatenate(confs, axis=-1)
    tps = [jnp.concatenate(tp, axis=-1) for tp in tplanes]

    iota = jnp.broadcast_to(jnp.arange(_TOTAL, dtype=jnp.int32)[None, :], conf.shape)
    negconf_s, idx_s, t0s, t1s, t2s, t3s = jax.lax.sort(
        (-conf, iota, tps[0], tps[1], tps[2], tps[3]),
        dimension=-1, is_stable=True, num_keys=1)
    vals = -negconf_s[:, :_TOPK]
    idx = idx_s[:, :_TOPK]
    npad = _PAD - _TOPK
    tg = [jnp.pad(t[:, :_TOPK], ((0, 0), (0, npad))).reshape(_B, 1, _ROWS, 128)
          for t in (t0s, t1s, t2s, t3s)]
    t_planes = jnp.concatenate(tg, axis=1)
    gidx = jnp.pad(idx, ((0, 0), (0, npad))).reshape(_B, _ROWS, 128)

    fi, planes = _nms(t_planes, gidx)
    fi = fi[:, :_OUT_P]
    conf_out = jnp.take_along_axis(vals, fi, axis=1)
    pf = planes.reshape(_B, 4, _PAD)
    box_out = jnp.stack([jnp.take_along_axis(pf[:, cc], fi, axis=1)
                         for cc in range(4)], axis=-1)
    return conf_out, box_out
